# baseline (speedup 1.0000x reference)
import functools

import numpy as np
import jax
import jax.numpy as jnp
from jax import lax
from jax.experimental import pallas as pl
from jax.experimental.pallas import tpu as pltpu
from jax.experimental.pallas import tpu_sc as plsc

B, C = 128, 512
R = 32
NCHUNK = C // R
OUT = C * (C + 1) // 2
SRC_W = R * C
L = 16

_iu0, _iu1 = np.triu_indices(C)
_rel_np = (_iu0 * C + _iu1 - (_iu0 // R) * (R * C)).astype(np.int32)
_OFFS = tuple(int(16400 * c - 512 * c * c) for c in range(NCHUNK))
_LENS = tuple(int(15888 - 1024 * c) for c in range(NCHUNK))

_LMAX = _LENS[0]


def _make_kernel():
    nc, ns = 2, 16
    nw = nc * ns
    bpw = B // nw
    mesh = plsc.VectorSubcoreMesh(core_axis_name="c", subcore_axis_name="s")

    @functools.partial(
        pl.kernel,
        mesh=mesh,
        out_type=jax.ShapeDtypeStruct((B * OUT,), jnp.float32),
        compiler_params=pltpu.CompilerParams(needs_layout_passes=False),
        scratch_types=[
            pltpu.VMEM((SRC_W,), jnp.float32),
            pltpu.VMEM((_LMAX,), jnp.int32),
            pltpu.VMEM((_LMAX,), jnp.float32),
        ],
    )
    def tri_kernel(x_hbm, rel_hbm, out_hbm, src_v, idx_v, pack_v):
        wid = lax.axis_index("s") * nc + lax.axis_index("c")
        b0 = wid * bpw

        def batch_body(t, _):
            b = b0 + t
            xb = pl.multiple_of(b * (C * C), 8)
            ob = pl.multiple_of(b * OUT, 8)
            for c in range(NCHUNK):
                ln = _LENS[c]
                off = _OFFS[c]
                pltpu.sync_copy(x_hbm.at[pl.ds(xb + c * SRC_W, SRC_W)],
                                src_v)
                pltpu.sync_copy(rel_hbm.at[pl.ds(off, ln)],
                                idx_v.at[pl.ds(0, ln)])

                def gather_body(k, _, _ln=ln):
                    base = k * L
                    iv = idx_v[pl.ds(base, L)]
                    pack_v[pl.ds(base, L)] = plsc.load_gather(src_v, [iv])
                    return ()

                lax.fori_loop(0, ln // L, gather_body, ())
                pltpu.sync_copy(pack_v.at[pl.ds(0, ln)],
                                out_hbm.at[pl.ds(ob + off, ln)])
            return ()

        lax.fori_loop(0, bpw, batch_body, ())

    return tri_kernel


_tri_cache = []


@jax.jit
def kernel(x):
    if not _tri_cache:
        _tri_cache.append(_make_kernel())
    x1 = x.reshape(B * C * C)
    return _tri_cache[0](x1, jnp.asarray(_rel_np)).reshape(B, OUT)


# speedup vs baseline: 1.6131x; 1.6131x over previous
import functools

import jax
import jax.numpy as jnp
from jax import lax
from jax.experimental import pallas as pl
from jax.experimental.pallas import tpu as pltpu
from jax.experimental.pallas import tpu_sc as plsc

B, C = 128, 512
R = 32
NCHUNK = C // R
OUT = C * (C + 1) // 2
CC = C * C
SRC_W = R * C
PACKW = 15888 + 16
L = 16

_OFFS = tuple(int(16400 * c - 512 * c * c) for c in range(NCHUNK))
_LENS = tuple(int(15888 - 1024 * c) for c in range(NCHUNK))


def _make_kernel():
    nc, ns = 2, 16
    nw = nc * ns
    bpw = B // nw
    mesh = plsc.VectorSubcoreMesh(core_axis_name="c", subcore_axis_name="s")

    @functools.partial(
        pl.kernel,
        mesh=mesh,
        out_type=jax.ShapeDtypeStruct((B * OUT,), jnp.float32),
        compiler_params=pltpu.CompilerParams(needs_layout_passes=False),
        scratch_types=[
            pltpu.VMEM((2 * SRC_W,), jnp.float32),
            pltpu.VMEM((bpw * PACKW,), jnp.float32),
            pltpu.SemaphoreType.DMA((2,)),
            pltpu.SemaphoreType.DMA((bpw,)),
        ],
    )
    def tri_kernel(x_hbm, out_hbm, src_v, pack_v, s_src, s_out):
        wid = lax.axis_index("s") * nc + lax.axis_index("c")
        b0 = wid * bpw

        def src_desc(c_base, t, p):
            xoff = pl.multiple_of((b0 + t) * CC + c_base, 8)
            voff = pl.multiple_of(p * SRC_W, 8)
            return pltpu.make_async_copy(
                x_hbm.at[pl.ds(xoff, SRC_W)],
                src_v.at[pl.ds(voff, SRC_W)],
                s_src.at[p])

        def out_desc(c, t):
            ooff = pl.multiple_of((b0 + t) * OUT + _OFFS[c], 8)
            poff = pl.multiple_of(t * PACKW, 8)
            return pltpu.make_async_copy(
                pack_v.at[pl.ds(poff, _LENS[c])],
                out_hbm.at[pl.ds(ooff, _LENS[c])],
                s_out.at[t])

        src_desc(0, 0, 0).start()

        for c in range(NCHUNK):
            i0 = R * c

            def chunk_body(t, _, c=c, i0=i0):
                p = lax.rem(t, 2)

                @pl.when(t < bpw - 1)
                def _():
                    src_desc(c * SRC_W, t + 1, lax.rem(t + 1, 2)).start()

                if c + 1 < NCHUNK:
                    @pl.when(t == bpw - 1)
                    def _():
                        src_desc((c + 1) * SRC_W, 0, 0).start()

                if c > 0:
                    out_desc(c - 1, t).wait()
                src_desc(c * SRC_W, t, p).wait()

                dbase = t * PACKW

                def row_body(r, carry):
                    a, d = carry
                    ln = (C - i0) - r
                    nblk = lax.shift_right_logical(ln + 15, 4)

                    def blk_body(u, _):
                        src = src_v[pl.ds(a + L * u, L)]
                        pack_v[pl.ds(d + L * u, L)] = src
                        return ()

                    lax.fori_loop(0, nblk, blk_body, ())
                    return (a + (C + 1), d + ln)

                lax.fori_loop(0, R, row_body,
                              (p * SRC_W + i0, dbase))

                out_desc(c, t).start()
                return ()

            lax.fori_loop(0, bpw, chunk_body, ())

        def drain_body(t, _):
            out_desc(NCHUNK - 1, t).wait()
            return ()

        lax.fori_loop(0, bpw, drain_body, ())

    return tri_kernel


_tri_cache = []


@jax.jit
def kernel(x):
    if not _tri_cache:
        _tri_cache.append(_make_kernel())
    x1 = x.reshape(B * CC)
    return _tri_cache[0](x1).reshape(B, OUT)


# speedup vs baseline: 2.0124x; 1.2475x over previous
import functools

import jax
import jax.numpy as jnp
from jax import lax
from jax.experimental import pallas as pl
from jax.experimental.pallas import tpu as pltpu
from jax.experimental.pallas import tpu_sc as plsc

B, C = 128, 512
R = 32
NCHUNK = C // R
OUT = C * (C + 1) // 2
CC = C * C
SRC_W = R * C
PACKW = 15888 + 16
L = 16

_OFFS = tuple(int(16400 * c - 512 * c * c) for c in range(NCHUNK))
_LENS = tuple(int(15888 - 1024 * c) for c in range(NCHUNK))


def _make_kernel():
    nc, ns = 2, 16
    nw = nc * ns
    bpw = B // nw
    mesh = plsc.VectorSubcoreMesh(core_axis_name="c", subcore_axis_name="s")

    @functools.partial(
        pl.kernel,
        mesh=mesh,
        out_type=jax.ShapeDtypeStruct((B * OUT,), jnp.float32),
        compiler_params=pltpu.CompilerParams(needs_layout_passes=False),
        scratch_types=[
            pltpu.VMEM((2 * SRC_W,), jnp.float32),
            pltpu.VMEM((bpw * PACKW,), jnp.float32),
            pltpu.SemaphoreType.DMA((2,)),
            pltpu.SemaphoreType.DMA((bpw,)),
        ],
    )
    def tri_kernel(x_hbm, out_hbm, src_v, pack_v, s_src, s_out):
        wid = lax.axis_index("s") * nc + lax.axis_index("c")
        b0 = wid * bpw

        def src_desc(c_base, t, p):
            xoff = pl.multiple_of((b0 + t) * CC + c_base, 8)
            voff = pl.multiple_of(p * SRC_W, 8)
            return pltpu.make_async_copy(
                x_hbm.at[pl.ds(xoff, SRC_W)],
                src_v.at[pl.ds(voff, SRC_W)],
                s_src.at[p])

        def out_desc(c, t):
            ooff = pl.multiple_of((b0 + t) * OUT + _OFFS[c], 8)
            poff = pl.multiple_of(t * PACKW, 8)
            return pltpu.make_async_copy(
                pack_v.at[pl.ds(poff, _LENS[c])],
                out_hbm.at[pl.ds(ooff, _LENS[c])],
                s_out.at[t])

        src_desc(0, 0, 0).start()

        for c in range(NCHUNK):
            i0 = R * c

            def chunk_body(t, _, c=c, i0=i0):
                p = lax.rem(t, 2)

                @pl.when(t < bpw - 1)
                def _():
                    src_desc(c * SRC_W, t + 1, lax.rem(t + 1, 2)).start()

                if c + 1 < NCHUNK:
                    @pl.when(t == bpw - 1)
                    def _():
                        src_desc((c + 1) * SRC_W, 0, 0).start()

                if c > 0:
                    out_desc(c - 1, t).wait()
                src_desc(c * SRC_W, t, p).wait()

                dbase = t * PACKW

                def row_body(r, carry):
                    a, d = carry
                    ln = (C - i0) - r
                    nblk = lax.shift_right_logical(ln + 15, 4)

                    @plsc.parallel_loop(0, nblk, 1, unroll=4)
                    def _(u):
                        pack_v[pl.ds(d + L * u, L)] = src_v[pl.ds(a + L * u, L)]
                    return (a + (C + 1), d + ln)

                lax.fori_loop(0, R, row_body,
                              (p * SRC_W + i0, dbase))

                out_desc(c, t).start()
                return ()

            lax.fori_loop(0, bpw, chunk_body, ())

        def drain_body(t, _):
            out_desc(NCHUNK - 1, t).wait()
            return ()

        lax.fori_loop(0, bpw, drain_body, ())

    return tri_kernel


_tri_cache = []


@jax.jit
def kernel(x):
    if not _tri_cache:
        _tri_cache.append(_make_kernel())
    x1 = x.reshape(B * CC)
    return _tri_cache[0](x1).reshape(B, OUT)


# speedup vs baseline: 2.5254x; 1.2549x over previous
import functools

import jax
import jax.numpy as jnp
from jax import lax
from jax.experimental import pallas as pl
from jax.experimental.pallas import tpu as pltpu
from jax.experimental.pallas import tpu_sc as plsc

B, C = 128, 512
R = 32
NCHUNK = C // R
OUT = C * (C + 1) // 2
CC = C * C
SRC_W = R * C
PACKW = 15888 + 16
L = 16

_OFFS = tuple(int(16400 * c - 512 * c * c) for c in range(NCHUNK))
_LENS = tuple(int(15888 - 1024 * c) for c in range(NCHUNK))


def _make_kernel():
    nc, ns = 2, 16
    nw = nc * ns
    bpw = B // nw
    mesh = plsc.VectorSubcoreMesh(core_axis_name="c", subcore_axis_name="s")

    @functools.partial(
        pl.kernel,
        mesh=mesh,
        out_type=jax.ShapeDtypeStruct((B * OUT,), jnp.float32),
        compiler_params=pltpu.CompilerParams(needs_layout_passes=False),
        scratch_types=[
            pltpu.VMEM((2 * R, C), jnp.float32),
            pltpu.VMEM((bpw * PACKW,), jnp.float32),
            pltpu.SemaphoreType.DMA((2,)),
            pltpu.SemaphoreType.DMA((bpw,)),
        ],
    )
    def tri_kernel(x_hbm, out_hbm, src_v, pack_v, s_src, s_out):
        wid = lax.axis_index("s") * nc + lax.axis_index("c")
        b0 = wid * bpw

        def src_desc(i0, t, p):
            voff = pl.multiple_of(p * R, 8)
            return pltpu.make_async_copy(
                x_hbm.at[b0 + t, pl.ds(i0, R), :],
                src_v.at[pl.ds(voff, R), :],
                s_src.at[p])

        def out_desc(c, t):
            ooff = pl.multiple_of((b0 + t) * OUT + _OFFS[c], 8)
            poff = pl.multiple_of(t * PACKW, 8)
            return pltpu.make_async_copy(
                pack_v.at[pl.ds(poff, _LENS[c])],
                out_hbm.at[pl.ds(ooff, _LENS[c])],
                s_out.at[t])

        src_desc(0, 0, 0).start()

        for c in range(NCHUNK):
            i0 = R * c

            def chunk_body(t, _, c=c, i0=i0):
                p = lax.rem(t, 2)

                @pl.when(t < bpw - 1)
                def _():
                    src_desc(i0, t + 1, lax.rem(t + 1, 2)).start()

                if c + 1 < NCHUNK:
                    @pl.when(t == bpw - 1)
                    def _():
                        src_desc(i0 + R, 0, 0).start()

                if c > 0:
                    out_desc(c - 1, t).wait()
                src_desc(i0, t, p).wait()

                dbase = t * PACKW
                rbase = p * R

                def row_body(k, d):
                    rr = (R - 1) - k
                    head = lax.rem(rr, L)
                    cb = (i0 + rr) - head
                    nblk = lax.shift_right_logical(C - cb, 4)
                    row = rbase + rr
                    dst0 = d - head

                    @plsc.parallel_loop(0, nblk, 1, unroll=4)
                    def _(u):
                        pack_v[pl.ds(dst0 + L * u, L)] = \
                            src_v[row, pl.ds(cb + L * u, L)]
                    return d - ((C - i0) - rr + 1)

                d31 = dbase + (R - 1) * (C - i0) - 465
                lax.fori_loop(0, R, row_body, d31)

                out_desc(c, t).start()
                return ()

            lax.fori_loop(0, bpw, chunk_body, ())

        def drain_body(t, _):
            out_desc(NCHUNK - 1, t).wait()
            return ()

        lax.fori_loop(0, bpw, drain_body, ())

    return tri_kernel


_tri_cache = []


@jax.jit
def kernel(x):
    if not _tri_cache:
        _tri_cache.append(_make_kernel())
    return _tri_cache[0](x).reshape(B, OUT)


# speedup vs baseline: 2.5322x; 1.0027x over previous
import functools

import jax
import jax.numpy as jnp
from jax import lax
from jax.experimental import pallas as pl
from jax.experimental.pallas import tpu as pltpu
from jax.experimental.pallas import tpu_sc as plsc

B, C = 128, 512
R = 32
NCHUNK = C // R
OUT = C * (C + 1) // 2
CC = C * C
SRC_W = R * C
PACKW = 15888 + 16
L = 16

_OFFS = tuple(int(16400 * c - 512 * c * c) for c in range(NCHUNK))
_LENS = tuple(int(15888 - 1024 * c) for c in range(NCHUNK))


def _make_kernel():
    nc, ns = 2, 16
    nw = nc * ns
    bpw = B // nw
    mesh = plsc.VectorSubcoreMesh(core_axis_name="c", subcore_axis_name="s")

    @functools.partial(
        pl.kernel,
        mesh=mesh,
        out_type=jax.ShapeDtypeStruct((B * OUT,), jnp.float32),
        compiler_params=pltpu.CompilerParams(needs_layout_passes=False),
        scratch_types=[
            pltpu.VMEM((2 * R, C), jnp.float32),
            pltpu.VMEM((bpw * PACKW,), jnp.float32),
            pltpu.SemaphoreType.DMA((2,)),
            pltpu.SemaphoreType.DMA((bpw,)),
        ],
    )
    def tri_kernel(x_hbm, out_hbm, src_v, pack_v, s_src, s_out):
        wid = lax.axis_index("s") * nc + lax.axis_index("c")
        b0 = wid * bpw

        def src_desc(i0, t, p):
            voff = pl.multiple_of(p * R, 8)
            return pltpu.make_async_copy(
                x_hbm.at[b0 + t, pl.ds(i0, R), :],
                src_v.at[pl.ds(voff, R), :],
                s_src.at[p])

        def out_desc(c, t):
            ooff = pl.multiple_of((b0 + t) * OUT + _OFFS[c], 8)
            poff = pl.multiple_of(t * PACKW, 8)
            return pltpu.make_async_copy(
                pack_v.at[pl.ds(poff, _LENS[c])],
                out_hbm.at[pl.ds(ooff, _LENS[c])],
                s_out.at[t])

        src_desc(0, 0, 0).start()

        for c in range(NCHUNK):
            i0 = R * c

            def chunk_body(t, _, c=c, i0=i0):
                p = lax.rem(t, 2)

                @pl.when(t < bpw - 1)
                def _():
                    src_desc(i0, t + 1, lax.rem(t + 1, 2)).start()

                if c + 1 < NCHUNK:
                    @pl.when(t == bpw - 1)
                    def _():
                        src_desc(i0 + R, 0, 0).start()

                if c > 0:
                    out_desc(c - 1, t).wait()
                src_desc(i0, t, p).wait()

                dbase = t * PACKW
                rbase = p * R

                def row_body(k, d):
                    rr = (R - 1) - k
                    head = lax.rem(rr, L)
                    cb = (i0 + rr) - head
                    nblk = lax.shift_right_logical(C - cb, 4)
                    row = rbase + rr
                    dst0 = d - head

                    @plsc.parallel_loop(0, nblk, 1, unroll=4)
                    def _(u):
                        pack_v[pl.ds(dst0 + L * u, L)] = \
                            src_v[row, pl.ds(cb + L * u, L)]
                    return d - ((C - i0) - rr + 1)

                d31 = dbase + (R - 1) * (C - i0) - 465
                lax.fori_loop(0, R, row_body, d31)

                out_desc(c, t).start()
                return ()

            lax.fori_loop(0, bpw, chunk_body, ())

        def drain_body(t, _):
            out_desc(NCHUNK - 1, t).wait()
            return ()

        lax.fori_loop(0, bpw, drain_body, ())

    return tri_kernel


_tri_cache = []


def _impl(x):
    if not _tri_cache:
        _tri_cache.append(_make_kernel())
    return _tri_cache[0](x).reshape(B, OUT)


_jit_cache = {}


def kernel(x):
    try:
        dev = next(iter(x.devices()))
    except Exception:
        dev = None
    fn = _jit_cache.get(dev)
    if fn is None:
        from jax._src.layout import Format, Layout
        from jax.sharding import SingleDeviceSharding
        if dev is None:
            fn = jax.jit(_impl)
        else:
            fmt = Format(Layout((0, 1), tiling=()),
                         SingleDeviceSharding(dev))
            fn = jax.jit(_impl, out_shardings=fmt)
        _jit_cache[dev] = fn
    return fn(x)


# speedup vs baseline: 2.5378x; 1.0022x over previous
import functools

import jax
import jax.numpy as jnp
from jax import lax
from jax.experimental import pallas as pl
from jax.experimental.pallas import tpu as pltpu
from jax.experimental.pallas import tpu_sc as plsc

B, C = 128, 512
R = 32
NCHUNK = C // R
OUT = C * (C + 1) // 2
CC = C * C
SRC_W = R * C
PACKW = 15888 + 16
L = 16

_OFFS = tuple(int(16400 * c - 512 * c * c) for c in range(NCHUNK))
_LENS = tuple(int(15888 - 1024 * c) for c in range(NCHUNK))


def _make_kernel():
    nc, ns = 2, 16
    nw = nc * ns
    bpw = B // nw
    mesh = plsc.VectorSubcoreMesh(core_axis_name="c", subcore_axis_name="s")

    @functools.partial(
        pl.kernel,
        mesh=mesh,
        out_type=jax.ShapeDtypeStruct((B * OUT,), jnp.float32),
        compiler_params=pltpu.CompilerParams(needs_layout_passes=False),
        scratch_types=[
            pltpu.VMEM((2 * R, C), jnp.float32),
            pltpu.VMEM((bpw * PACKW,), jnp.float32),
            pltpu.SemaphoreType.DMA((2,)),
            pltpu.SemaphoreType.DMA((bpw,)),
        ],
    )
    def tri_kernel(x_hbm, out_hbm, src_v, pack_v, s_src, s_out):
        wid = lax.axis_index("s") * nc + lax.axis_index("c")
        b0 = wid * bpw

        def src_desc(i0, t, p):
            voff = pl.multiple_of(p * R, 8)
            return pltpu.make_async_copy(
                x_hbm.at[b0 + t, pl.ds(i0, R), :],
                src_v.at[pl.ds(voff, R), :],
                s_src.at[p])

        def out_desc(c, t):
            ooff = pl.multiple_of((b0 + t) * OUT + _OFFS[c], 8)
            poff = pl.multiple_of(t * PACKW, 8)
            return pltpu.make_async_copy(
                pack_v.at[pl.ds(poff, _LENS[c])],
                out_hbm.at[pl.ds(ooff, _LENS[c])],
                s_out.at[t])

        src_desc(0, 0, 0).start()

        for c in range(NCHUNK):
            i0 = R * c

            def chunk_body(t, _, c=c, i0=i0):
                p = lax.rem(t, 2)

                @pl.when(t < bpw - 1)
                def _():
                    src_desc(i0, t + 1, lax.rem(t + 1, 2)).start()

                if c + 1 < NCHUNK:
                    @pl.when(t == bpw - 1)
                    def _():
                        src_desc(i0 + R, 0, 0).start()

                if c > 0:
                    out_desc(c - 1, t).wait()
                src_desc(i0, t, p).wait()

                dbase = t * PACKW
                rbase = p * R

                def row_body(k, d):
                    rr = (R - 1) - k
                    head = lax.rem(rr, L)
                    cb = (i0 + rr) - head
                    nblk = lax.shift_right_logical(C - cb, 4)
                    row = rbase + rr
                    dst0 = d - head

                    @plsc.parallel_loop(0, nblk, 1, unroll=4)
                    def _(u):
                        pack_v[pl.ds(dst0 + L * u, L)] = \
                            src_v[row, pl.ds(cb + L * u, L)]
                    return d - ((C - i0) - rr + 1)

                d31 = dbase + (R - 1) * (C - i0) - 465
                lax.fori_loop(0, R, row_body, d31)

                out_desc(c, t).start()
                return ()

            lax.fori_loop(0, bpw, chunk_body, ())

        def drain_body(t, _):
            out_desc(NCHUNK - 1, t).wait()
            return ()

        lax.fori_loop(0, bpw, drain_body, ())

    return tri_kernel


_tri_cache = []


def _impl(x):
    if not _tri_cache:
        _tri_cache.append(_make_kernel())
    return _tri_cache[0](x).reshape(B, OUT)


_jit_cache = {}


def kernel(x):
    try:
        dev = next(iter(x.devices()))
    except Exception:
        try:
            dev = jax.devices()[0]
        except Exception:
            dev = None
    fn = _jit_cache.get(dev)
    if fn is None:
        from jax._src.layout import Format, Layout
        from jax.sharding import SingleDeviceSharding
        if dev is None:
            fn = jax.jit(_impl)
        else:
            fmt = Format(Layout((0, 1), tiling=()),
                         SingleDeviceSharding(dev))
            fn = jax.jit(_impl, out_shardings=fmt)
        _jit_cache[dev] = fn
    return fn(x)
